# Initial kernel scaffold; baseline (speedup 1.0000x reference)
import jax
import jax.numpy as jnp
from jax.experimental import pallas as pl


def kernel(x, edge_index, batch, hom_mask, het_mask, coe, W_hom0, b_hom0, W_het0, b_het0, W_hom1, b_hom1, W_het1, b_het1, W1, b1, W2, b2, W3, b3):
    raise NotImplementedError("write your pallas kernel here")


# speedup vs baseline: 4.3967x; 4.3967x over previous
import functools
import math

import jax
import jax.numpy as jnp
import numpy as np
from jax import lax
from jax.experimental import pallas as pl
from jax.experimental.pallas import tpu as pltpu
from jax.experimental.pallas import tpu_sc as plsc

N = 10000
E = 160000
K = 10
G = 64

_NC = 2
_NS = 16
_NW = _NC * _NS
_CHUNK = 128
_CHUNKS = 40
_E_PAD = _NW * _CHUNKS * _CHUNK
_ROWS_PER_TILE = 640
_ROWS_PAD = _NS * _ROWS_PER_TILE
_TRASH = N

_CMAT = np.zeros((K + 1, K + 1), dtype=np.float64)
for _j in range(K + 1):
    for _m in range(K + 1):
        _CMAT[_j, _m] = sum(
            (-1) ** _p * math.comb(_j, _p) * math.comb(K - _j, _m - _p)
            for _p in range(max(0, _m - (K - _j)), min(_j, _m) + 1)
        )
_CMAT = _CMAT.astype(np.float32)
_BINOM = np.asarray(
    [math.comb(K, j) / (2.0 ** K) for j in range(K + 1)], dtype=np.float32
)


def _prop_body(q_hbm, row_hbm, col_hbm, zero_hbm, out_hbm,
               row_v, col_v, gbuf, acc_sh, sem):
    c = lax.axis_index("c")
    s = lax.axis_index("s")
    wid = s * _NC + c
    pltpu.sync_copy(row_hbm.at[wid], row_v)
    pltpu.sync_copy(col_hbm.at[wid], col_v)
    pltpu.sync_copy(zero_hbm, acc_sh.at[pl.ds(s * _ROWS_PER_TILE, _ROWS_PER_TILE)])
    plsc.subcore_barrier()

    def step(j, carry):
        pltpu.async_copy(q_hbm.at[row_v.at[j]], gbuf, sem).wait()
        pltpu.sync_copy(gbuf, acc_sh.at[col_v.at[j]], add=True)
        return carry

    lax.fori_loop(0, _CHUNKS, step, 0)
    plsc.subcore_barrier()
    pltpu.sync_copy(acc_sh.at[pl.ds(s * _ROWS_PER_TILE, _ROWS_PER_TILE)],
                    out_hbm.at[c].at[pl.ds(s * _ROWS_PER_TILE, _ROWS_PER_TILE)])


def _sc_prop(q, row_idx, col_idx, zeros):
    kern = pl.kernel(
        _prop_body,
        out_type=jax.ShapeDtypeStruct((_NC, _ROWS_PAD, 128), jnp.float32),
        mesh=plsc.VectorSubcoreMesh(
            core_axis_name="c", subcore_axis_name="s",
            num_cores=_NC, num_subcores=_NS),
        scratch_types=[
            pltpu.VMEM((_CHUNKS, _CHUNK), jnp.int32),
            pltpu.VMEM((_CHUNKS, _CHUNK), jnp.int32),
            pltpu.VMEM((_CHUNK, 128), jnp.float32),
            pltpu.VMEM_SHARED((_ROWS_PAD, 128), jnp.float32),
            pltpu.SemaphoreType.DMA,
        ],
    )
    return kern(q, row_idx, col_idx, zeros)


def _lin_body(x_ref, w_ref, b_ref, o_ref):
    o_ref[...] = (
        jnp.dot(x_ref[...], w_ref[...], preferred_element_type=jnp.float32)
        + b_ref[...]
    )


def _lin(x, W, b):
    n, fin = x.shape
    fout = W.shape[1]
    bm = 1000
    return pl.pallas_call(
        _lin_body,
        grid=(n // bm,),
        in_specs=[
            pl.BlockSpec((bm, fin), lambda i: (i, 0)),
            pl.BlockSpec((fin, fout), lambda i: (0, 0)),
            pl.BlockSpec((1, fout), lambda i: (0, 0)),
        ],
        out_specs=pl.BlockSpec((bm, fout), lambda i: (i, 0)),
        out_shape=jax.ShapeDtypeStruct((n, fout), jnp.float32),
    )(x, W, b.reshape(1, fout))


_RT = 1000
_NEG = -3.0e38


def _readout_body(h1_ref, h2_ref, b_ref, bc_ref, w1_ref, b1_ref, w2_ref,
                  b2_ref, w3_ref, b3_ref, o_ref, s1, s2, m1, m2, cnt):
    i = pl.program_id(0)

    @pl.when(i == 0)
    def _init():
        s1[...] = jnp.zeros_like(s1)
        s2[...] = jnp.zeros_like(s2)
        m1[...] = jnp.full_like(m1, _NEG)
        m2[...] = jnp.full_like(m2, _NEG)
        cnt[...] = jnp.zeros_like(cnt)

    h1 = h1_ref[...]
    h2 = h2_ref[...]
    brow = b_ref[0]
    bc = bc_ref[...]
    gids = lax.broadcasted_iota(jnp.int32, (G, _RT), 0)
    msk = (brow == gids).astype(jnp.float32)
    s1[...] += jnp.dot(msk, h1, preferred_element_type=jnp.float32)
    s2[...] += jnp.dot(msk, h2, preferred_element_type=jnp.float32)
    cnt[...] += jnp.dot(msk, jnp.ones((_RT, 128), jnp.float32),
                        preferred_element_type=jnp.float32)

    sel_col = lax.broadcasted_iota(jnp.int32, (G, 1), 0)

    def gstep(g, carry):
        mg = bc == g.astype(jnp.float32)
        t1 = jnp.max(jnp.where(mg, h1, _NEG), axis=0, keepdims=True)
        t2 = jnp.max(jnp.where(mg, h2, _NEG), axis=0, keepdims=True)
        sel = sel_col == g
        m1[...] = jnp.where(sel, jnp.maximum(m1[...], t1), m1[...])
        m2[...] = jnp.where(sel, jnp.maximum(m2[...], t2), m2[...])
        return carry

    lax.fori_loop(0, G, gstep, 0)

    @pl.when(i == pl.num_programs(0) - 1)
    def _final():
        c = cnt[...][:, 0:1]
        mx = jnp.where(c > 0, m1[...] + m2[...], 0.0)
        mean = (s1[...] + s2[...]) / jnp.maximum(c, 1.0)
        z = jnp.concatenate([mx, mean], axis=1)
        a = jnp.maximum(
            jnp.dot(z, w1_ref[...], preferred_element_type=jnp.float32)
            + b1_ref[...], 0.0)
        a = jnp.maximum(
            jnp.dot(a, w2_ref[...], preferred_element_type=jnp.float32)
            + b2_ref[...], 0.0)
        l = (jnp.dot(a, w3_ref[...], preferred_element_type=jnp.float32)
             + b3_ref[...])
        mlog = jnp.max(l, axis=1, keepdims=True)
        lse = jnp.log(jnp.sum(jnp.exp(l - mlog), axis=1, keepdims=True)) + mlog
        o_ref[...] = l - lse


def _readout_mlp(h1, h2, batch, W1, b1, W2, b2, W3, b3):
    grid = N // _RT
    batch3 = batch.astype(jnp.int32).reshape(grid, 1, _RT)
    batch_bc = jnp.broadcast_to(batch.astype(jnp.float32)[:, None], (N, 256))
    W3p = jnp.pad(W3, ((0, 0), (0, 128 - W3.shape[1])))
    b3p = jnp.pad(b3, (0, 128 - b3.shape[0]), constant_values=-1.0e30)
    full = lambda shape: pl.BlockSpec(shape, lambda i: tuple(0 for _ in shape))
    return pl.pallas_call(
        _readout_body,
        grid=(grid,),
        in_specs=[
            pl.BlockSpec((_RT, 256), lambda i: (i, 0)),
            pl.BlockSpec((_RT, 256), lambda i: (i, 0)),
            pl.BlockSpec((1, 1, _RT), lambda i: (i, 0, 0)),
            pl.BlockSpec((_RT, 256), lambda i: (i, 0)),
            full((512, 256)), full((1, 256)),
            full((256, 128)), full((1, 128)),
            full((128, 128)), full((1, 128)),
        ],
        out_specs=full((G, 128)),
        out_shape=jax.ShapeDtypeStruct((G, 128), jnp.float32),
        scratch_shapes=[
            pltpu.VMEM((G, 256), jnp.float32),
            pltpu.VMEM((G, 256), jnp.float32),
            pltpu.VMEM((G, 256), jnp.float32),
            pltpu.VMEM((G, 256), jnp.float32),
            pltpu.VMEM((G, 128), jnp.float32),
        ],
    )(h1, h2, batch3, batch_bc, W1, b1.reshape(1, -1), W2, b2.reshape(1, -1),
      W3p, b3p.reshape(1, -1))


def _pad_idx(v, fill):
    pad = jnp.full((_E_PAD - E,), fill, dtype=jnp.int32)
    return jnp.concatenate([v.astype(jnp.int32), pad]).reshape(
        _NW, _CHUNKS, _CHUNK)


def kernel(x, edge_index, batch, hom_mask, het_mask, coe,
           W_hom0, b_hom0, W_het0, b_het0, W_hom1, b_hom1, W_het1, b_het1,
           W1, b1, W2, b2, W3, b3):
    row = edge_index[0]
    col = edge_index[1]
    alpha = (jax.nn.relu(coe) * _BINOM) @ _CMAT

    zeros640 = jnp.zeros((_ROWS_PER_TILE, 128), jnp.float32)
    ones8 = jnp.ones((8, 128), jnp.float32)
    zidx = jnp.zeros((_NW, _CHUNKS, _CHUNK), jnp.int32)

    row_p = _pad_idx(row, 0)

    def view(mask):
        colv = _pad_idx(jnp.where(mask, col, _TRASH), _TRASH)
        degcol = _pad_idx(jnp.where(mask, row, _TRASH), _TRASH)
        o = _sc_prop(ones8, zidx, degcol, zeros640)
        deg = o[0, :N, 0] + o[1, :N, 0]
        d = jnp.where(deg > 0, lax.rsqrt(deg), 0.0)
        return colv, d[:, None]

    colh, dh = view(hom_mask)
    colt, dt = view(het_mask)

    def conv(feat, W, b, d, colv):
        h = _lin(feat, W, b)
        acc = alpha[0] * h
        g = d * h
        for j in range(1, K + 1):
            o = _sc_prop(g, row_p, colv, zeros640)
            p = d * (o[0, :N] + o[1, :N])
            acc = acc + alpha[j] * p
            if j < K:
                g = d * p
        return acc

    xh = jax.nn.relu(conv(x, W_hom0, b_hom0, dh, colh))
    xt = jax.nn.relu(conv(x, W_het0, b_het0, dt, colt))
    h1 = jnp.concatenate([xh, xt], axis=1)
    xh = jax.nn.relu(conv(h1, W_hom1, b_hom1, dh, colh))
    xt = jax.nn.relu(conv(h1, W_het1, b_het1, dt, colt))
    h2 = jnp.concatenate([xh, xt], axis=1)

    out = _readout_mlp(h1, h2, batch, W1, b1, W2, b2, W3, b3)
    return out[:, :10]


# speedup vs baseline: 9.0811x; 2.0655x over previous
import functools
import math

import jax
import jax.numpy as jnp
import numpy as np
from jax import lax
from jax.experimental import pallas as pl
from jax.experimental.pallas import tpu as pltpu
from jax.experimental.pallas import tpu_sc as plsc

N = 10000
E = 160000
K = 10
G = 64

_NC = 2
_NS = 16
_NW = _NC * _NS
_CHUNK = 128
_CHUNKS = 40
_E_PAD = _NW * _CHUNKS * _CHUNK
_ROWS_PER_TILE = 640
_ROWS_PAD = _NS * _ROWS_PER_TILE
_TRASH = N

_CMAT = np.zeros((K + 1, K + 1), dtype=np.float64)
for _j in range(K + 1):
    for _m in range(K + 1):
        _CMAT[_j, _m] = sum(
            (-1) ** _p * math.comb(_j, _p) * math.comb(K - _j, _m - _p)
            for _p in range(max(0, _m - (K - _j)), min(_j, _m) + 1)
        )
_CMAT = _CMAT.astype(np.float32)
_BINOM = np.asarray(
    [math.comb(K, j) / (2.0 ** K) for j in range(K + 1)], dtype=np.float32
)


def _prop_body(q_hbm, row_hbm, col_hbm, zero_hbm, out_hbm,
               row_v, col_v, g0, g1, acc_sh, sem0, sem1):
    c = lax.axis_index("c")
    s = lax.axis_index("s")
    wid = s * _NC + c
    pltpu.sync_copy(row_hbm.at[wid], row_v)
    pltpu.sync_copy(col_hbm.at[wid], col_v)
    pltpu.sync_copy(zero_hbm, acc_sh.at[pl.ds(s * _ROWS_PER_TILE, _ROWS_PER_TILE)])
    plsc.subcore_barrier()

    pltpu.async_copy(q_hbm.at[row_v.at[0]], g0, sem0)
    pltpu.async_copy(q_hbm.at[row_v.at[1]], g1, sem1)

    def step(i, carry):
        j0 = 2 * i
        pltpu.make_async_copy(q_hbm.at[row_v.at[j0]], g0, sem0).wait()
        pltpu.sync_copy(g0, acc_sh.at[col_v.at[j0]], add=True)

        @pl.when(i < _CHUNKS // 2 - 1)
        def _f0():
            pltpu.async_copy(q_hbm.at[row_v.at[j0 + 2]], g0, sem0)

        pltpu.make_async_copy(q_hbm.at[row_v.at[j0 + 1]], g1, sem1).wait()
        pltpu.sync_copy(g1, acc_sh.at[col_v.at[j0 + 1]], add=True)

        @pl.when(i < _CHUNKS // 2 - 1)
        def _f1():
            pltpu.async_copy(q_hbm.at[row_v.at[j0 + 3]], g1, sem1)

        return carry

    lax.fori_loop(0, _CHUNKS // 2, step, 0)
    plsc.subcore_barrier()
    pltpu.sync_copy(acc_sh.at[pl.ds(s * _ROWS_PER_TILE, _ROWS_PER_TILE)],
                    out_hbm.at[c].at[pl.ds(s * _ROWS_PER_TILE, _ROWS_PER_TILE)])


def _sc_prop(q, row_idx, col_idx, zeros):
    kern = pl.kernel(
        _prop_body,
        out_type=jax.ShapeDtypeStruct((_NC, _ROWS_PAD, 128), jnp.float32),
        mesh=plsc.VectorSubcoreMesh(
            core_axis_name="c", subcore_axis_name="s",
            num_cores=_NC, num_subcores=_NS),
        scratch_types=[
            pltpu.VMEM((_CHUNKS, _CHUNK), jnp.int32),
            pltpu.VMEM((_CHUNKS, _CHUNK), jnp.int32),
            pltpu.VMEM((_CHUNK, 128), jnp.float32),
            pltpu.VMEM((_CHUNK, 128), jnp.float32),
            pltpu.VMEM_SHARED((_ROWS_PAD, 128), jnp.float32),
            pltpu.SemaphoreType.DMA,
            pltpu.SemaphoreType.DMA,
        ],
    )
    return kern(q, row_idx, col_idx, zeros)


def _lin_body(x_ref, w_ref, b_ref, o_ref):
    o_ref[...] = (
        jnp.dot(x_ref[...], w_ref[...], preferred_element_type=jnp.float32)
        + b_ref[...]
    )


def _lin(x, W, b):
    n, fin = x.shape
    fout = W.shape[1]
    bm = 1000
    return pl.pallas_call(
        _lin_body,
        grid=(n // bm,),
        in_specs=[
            pl.BlockSpec((bm, fin), lambda i: (i, 0)),
            pl.BlockSpec((fin, fout), lambda i: (0, 0)),
            pl.BlockSpec((1, fout), lambda i: (0, 0)),
        ],
        out_specs=pl.BlockSpec((bm, fout), lambda i: (i, 0)),
        out_shape=jax.ShapeDtypeStruct((n, fout), jnp.float32),
    )(x, W, b.reshape(1, fout))


_RT = 1000
_NEG = -3.0e38


def _readout_body(h1_ref, h2_ref, b_ref, bc_ref, w1_ref, b1_ref, w2_ref,
                  b2_ref, w3_ref, b3_ref, o_ref, s1, s2, m1, m2, cnt):
    i = pl.program_id(0)

    @pl.when(i == 0)
    def _init():
        s1[...] = jnp.zeros_like(s1)
        s2[...] = jnp.zeros_like(s2)
        m1[...] = jnp.full_like(m1, _NEG)
        m2[...] = jnp.full_like(m2, _NEG)
        cnt[...] = jnp.zeros_like(cnt)

    h1 = h1_ref[...]
    h2 = h2_ref[...]
    brow = b_ref[0]
    bc = bc_ref[...]
    gids = lax.broadcasted_iota(jnp.int32, (G, _RT), 0)
    msk = (brow == gids).astype(jnp.float32)
    s1[...] += jnp.dot(msk, h1, preferred_element_type=jnp.float32)
    s2[...] += jnp.dot(msk, h2, preferred_element_type=jnp.float32)
    cnt[...] += jnp.dot(msk, jnp.ones((_RT, 128), jnp.float32),
                        preferred_element_type=jnp.float32)

    sel_col = lax.broadcasted_iota(jnp.int32, (G, 1), 0)

    def gstep(g, carry):
        mg = bc == g.astype(jnp.float32)
        t1 = jnp.max(jnp.where(mg, h1, _NEG), axis=0, keepdims=True)
        t2 = jnp.max(jnp.where(mg, h2, _NEG), axis=0, keepdims=True)
        sel = sel_col == g
        m1[...] = jnp.where(sel, jnp.maximum(m1[...], t1), m1[...])
        m2[...] = jnp.where(sel, jnp.maximum(m2[...], t2), m2[...])
        return carry

    lax.fori_loop(0, G, gstep, 0)

    @pl.when(i == pl.num_programs(0) - 1)
    def _final():
        c = cnt[...][:, 0:1]
        mx = jnp.where(c > 0, m1[...] + m2[...], 0.0)
        mean = (s1[...] + s2[...]) / jnp.maximum(c, 1.0)
        z = jnp.concatenate([mx, mean], axis=1)
        a = jnp.maximum(
            jnp.dot(z, w1_ref[...], preferred_element_type=jnp.float32)
            + b1_ref[...], 0.0)
        a = jnp.maximum(
            jnp.dot(a, w2_ref[...], preferred_element_type=jnp.float32)
            + b2_ref[...], 0.0)
        l = (jnp.dot(a, w3_ref[...], preferred_element_type=jnp.float32)
             + b3_ref[...])
        mlog = jnp.max(l, axis=1, keepdims=True)
        lse = jnp.log(jnp.sum(jnp.exp(l - mlog), axis=1, keepdims=True)) + mlog
        o_ref[...] = l - lse


def _readout_mlp(h1, h2, batch, W1, b1, W2, b2, W3, b3):
    grid = N // _RT
    batch3 = batch.astype(jnp.int32).reshape(grid, 1, _RT)
    batch_bc = jnp.broadcast_to(batch.astype(jnp.float32)[:, None], (N, 256))
    W3p = jnp.pad(W3, ((0, 0), (0, 128 - W3.shape[1])))
    b3p = jnp.pad(b3, (0, 128 - b3.shape[0]), constant_values=-1.0e30)
    full = lambda shape: pl.BlockSpec(shape, lambda i: tuple(0 for _ in shape))
    return pl.pallas_call(
        _readout_body,
        grid=(grid,),
        in_specs=[
            pl.BlockSpec((_RT, 256), lambda i: (i, 0)),
            pl.BlockSpec((_RT, 256), lambda i: (i, 0)),
            pl.BlockSpec((1, 1, _RT), lambda i: (i, 0, 0)),
            pl.BlockSpec((_RT, 256), lambda i: (i, 0)),
            full((512, 256)), full((1, 256)),
            full((256, 128)), full((1, 128)),
            full((128, 128)), full((1, 128)),
        ],
        out_specs=full((G, 128)),
        out_shape=jax.ShapeDtypeStruct((G, 128), jnp.float32),
        scratch_shapes=[
            pltpu.VMEM((G, 256), jnp.float32),
            pltpu.VMEM((G, 256), jnp.float32),
            pltpu.VMEM((G, 256), jnp.float32),
            pltpu.VMEM((G, 256), jnp.float32),
            pltpu.VMEM((G, 128), jnp.float32),
        ],
    )(h1, h2, batch3, batch_bc, W1, b1.reshape(1, -1), W2, b2.reshape(1, -1),
      W3p, b3p.reshape(1, -1))


def _pad_idx(v, fill):
    pad = jnp.full((_E_PAD - E,), fill, dtype=jnp.int32)
    return jnp.concatenate([v.astype(jnp.int32), pad]).reshape(
        _NW, _CHUNKS, _CHUNK)


def kernel(x, edge_index, batch, hom_mask, het_mask, coe,
           W_hom0, b_hom0, W_het0, b_het0, W_hom1, b_hom1, W_het1, b_het1,
           W1, b1, W2, b2, W3, b3):
    row = edge_index[0]
    col = edge_index[1]
    alpha = (jax.nn.relu(coe) * _BINOM) @ _CMAT

    zeros640 = jnp.zeros((_ROWS_PER_TILE, 128), jnp.float32)
    ones_tab = jnp.ones((_CHUNK, 128), jnp.float32)
    lane_idx = jnp.broadcast_to(
        jnp.arange(_CHUNK, dtype=jnp.int32), (_NW, _CHUNKS, _CHUNK))

    row_p = _pad_idx(row, 0)

    def view(mask):
        colv = _pad_idx(jnp.where(mask, col, _TRASH), _TRASH)
        degcol = _pad_idx(jnp.where(mask, row, _TRASH), _TRASH)
        o = _sc_prop(ones_tab, lane_idx, degcol, zeros640)
        deg = o[0, :N, 0] + o[1, :N, 0]
        d = jnp.where(deg > 0, lax.rsqrt(deg), 0.0)
        return colv, d[:, None]

    colh, dh = view(hom_mask)
    colt, dt = view(het_mask)

    def conv(feat, W, b, d, colv):
        h = _lin(feat, W, b)
        acc = alpha[0] * h
        g = d * h
        for j in range(1, K + 1):
            o = _sc_prop(g, row_p, colv, zeros640)
            p = d * (o[0, :N] + o[1, :N])
            acc = acc + alpha[j] * p
            if j < K:
                g = d * p
        return acc

    xh = jax.nn.relu(conv(x, W_hom0, b_hom0, dh, colh))
    xt = jax.nn.relu(conv(x, W_het0, b_het0, dt, colt))
    h1 = jnp.concatenate([xh, xt], axis=1)
    xh = jax.nn.relu(conv(h1, W_hom1, b_hom1, dh, colh))
    xt = jax.nn.relu(conv(h1, W_het1, b_het1, dt, colt))
    h2 = jnp.concatenate([xh, xt], axis=1)

    out = _readout_mlp(h1, h2, batch, W1, b1, W2, b2, W3, b3)
    return out[:, :10]
